# baseline (speedup 1.0000x reference)
import functools

import jax
import jax.numpy as jnp
from jax import lax
from jax.experimental import pallas as pl
from jax.experimental.pallas import tpu as pltpu, tpu_sc as plsc

N = 10000
E = 160000
DN = 256
DE = 16
LAT = 512

NC = 2
NS = 16
NW = NC * NS
CHUNK = 128
K = -(-E // (NW * CHUNK))
E_PAD = NW * K * CHUNK
STRIPE = 8 * (-(-N // (NS * 8)))
N_ACC = NS * STRIPE

BE = 512
BN = 512
N_PAD = -(-N // BN) * BN


def _sc_gather_scatter(node_tab, sidx3, didx3, edge_pad, zeros_acc):
    mesh = plsc.VectorSubcoreMesh(core_axis_name="c", subcore_axis_name="s")

    @functools.partial(
        pl.kernel,
        mesh=mesh,
        compiler_params=pltpu.CompilerParams(use_tc_tiling_on_sc=False),
        out_type=(
            jax.ShapeDtypeStruct((E_PAD, DN), node_tab.dtype),
            jax.ShapeDtypeStruct((E_PAD, DN), node_tab.dtype),
            jax.ShapeDtypeStruct((NC, N_ACC, DE), jnp.float32),
        ),
        scratch_types=[
            pltpu.VMEM((K, CHUNK), jnp.int32),
            pltpu.VMEM((K, CHUNK), jnp.int32),
            pltpu.VMEM((CHUNK, DN), node_tab.dtype),
            pltpu.VMEM((CHUNK, DN), node_tab.dtype),
            pltpu.VMEM((CHUNK, DE), jnp.float32),
            pltpu.VMEM((STRIPE, DE), jnp.float32),
            pltpu.VMEM_SHARED((N_ACC, DE), jnp.float32),
            pltpu.SemaphoreType.DMA,
        ],
    )
    def kern(node_hbm, sidx_hbm, didx_hbm, edge_hbm, zeros_hbm,
             gsrc_hbm, gdst_hbm, psum_hbm,
             idx_s, idx_d, rows_a, rows_b, erows, sbuf, acc, sem):
        c = lax.axis_index("c")
        s = lax.axis_index("s")
        wid = s * NC + c
        base = wid * (K * CHUNK)

        pltpu.sync_copy(sidx_hbm.at[wid], idx_s)
        pltpu.sync_copy(didx_hbm.at[wid], idx_d)
        pltpu.sync_copy(zeros_hbm.at[pl.ds(s * STRIPE, STRIPE)], sbuf)
        pltpu.sync_copy(sbuf, acc.at[pl.ds(s * STRIPE, STRIPE)])
        plsc.subcore_barrier()

        def body(j, carry):
            off = base + j * CHUNK
            pltpu.async_copy(node_hbm.at[idx_s.at[j]], rows_a, sem).wait()
            pltpu.sync_copy(rows_a, gsrc_hbm.at[pl.ds(off, CHUNK)])
            pltpu.async_copy(node_hbm.at[idx_d.at[j]], rows_b, sem).wait()
            pltpu.sync_copy(rows_b, gdst_hbm.at[pl.ds(off, CHUNK)])
            pltpu.sync_copy(edge_hbm.at[pl.ds(off, CHUNK)], erows)
            pltpu.sync_copy(erows, acc.at[idx_d.at[j]], add=True)
            return carry

        lax.fori_loop(0, K, body, 0)
        plsc.subcore_barrier()
        pltpu.sync_copy(acc.at[pl.ds(s * STRIPE, STRIPE)], sbuf)
        pltpu.sync_copy(sbuf, psum_hbm.at[c, pl.ds(s * STRIPE, STRIPE)])

    return kern(node_tab, sidx3, didx3, edge_pad, zeros_acc)


def _edge_mlp(gsrc, gdst, edge_pad, w1s, w1d, w1x, w2, g, b):
    def body(gs, gd, ef, w1s_r, w1d_r, w1x_r, w2_r, g_r, b_r, out):
        h = jnp.dot(gs[...], w1s_r[...], preferred_element_type=jnp.float32)
        h = h + jnp.dot(gd[...], w1d_r[...], preferred_element_type=jnp.float32)
        h = h + jnp.dot(ef[...], w1x_r[...], preferred_element_type=jnp.float32)
        h = h * jax.nn.sigmoid(h)
        u = jnp.dot(h, w2_r[...], preferred_element_type=jnp.float32)
        mu = jnp.mean(u, axis=-1, keepdims=True)
        var = jnp.mean((u - mu) * (u - mu), axis=-1, keepdims=True)
        y = (u - mu) * lax.rsqrt(var + 1e-5) * g_r[...] + b_r[...]
        out[...] = y + ef[...]

    grid = (E_PAD // BE,)
    return pl.pallas_call(
        body,
        grid=grid,
        in_specs=[
            pl.BlockSpec((BE, DN), lambda i: (i, 0)),
            pl.BlockSpec((BE, DN), lambda i: (i, 0)),
            pl.BlockSpec((BE, DE), lambda i: (i, 0)),
            pl.BlockSpec((DN, LAT), lambda i: (0, 0)),
            pl.BlockSpec((DN, LAT), lambda i: (0, 0)),
            pl.BlockSpec((DE, LAT), lambda i: (0, 0)),
            pl.BlockSpec((LAT, DE), lambda i: (0, 0)),
            pl.BlockSpec((1, DE), lambda i: (0, 0)),
            pl.BlockSpec((1, DE), lambda i: (0, 0)),
        ],
        out_specs=pl.BlockSpec((BE, DE), lambda i: (i, 0)),
        out_shape=jax.ShapeDtypeStruct((E_PAD, DE), jnp.float32),
    )(gsrc, gdst, edge_pad, w1s, w1d, w1x, w2, g, b)


def _node_mlp(nf_pad, p0, p1, w1nn, w1ne, w2, g, b):
    def body(nf, p0_r, p1_r, w1nn_r, w1ne_r, w2_r, g_r, b_r, out):
        se = p0_r[...] + p1_r[...]
        h = jnp.dot(nf[...], w1nn_r[...], preferred_element_type=jnp.float32)
        h = h + jnp.dot(se, w1ne_r[...], preferred_element_type=jnp.float32)
        h = h * jax.nn.sigmoid(h)
        u = jnp.dot(h, w2_r[...], preferred_element_type=jnp.float32)
        mu = jnp.mean(u, axis=-1, keepdims=True)
        var = jnp.mean((u - mu) * (u - mu), axis=-1, keepdims=True)
        y = (u - mu) * lax.rsqrt(var + 1e-5) * g_r[...] + b_r[...]
        out[...] = y + nf[...]

    grid = (N_PAD // BN,)
    return pl.pallas_call(
        body,
        grid=grid,
        in_specs=[
            pl.BlockSpec((BN, DN), lambda i: (i, 0)),
            pl.BlockSpec((BN, DE), lambda i: (i, 0)),
            pl.BlockSpec((BN, DE), lambda i: (i, 0)),
            pl.BlockSpec((DN, LAT), lambda i: (0, 0)),
            pl.BlockSpec((DE, LAT), lambda i: (0, 0)),
            pl.BlockSpec((LAT, DN), lambda i: (0, 0)),
            pl.BlockSpec((1, DN), lambda i: (0, 0)),
            pl.BlockSpec((1, DN), lambda i: (0, 0)),
        ],
        out_specs=pl.BlockSpec((BN, DN), lambda i: (i, 0)),
        out_shape=jax.ShapeDtypeStruct((N_PAD, DN), jnp.float32),
    )(nf_pad, p0, p1, w1nn, w1ne, w2, g, b)


def kernel(node_feats, edge_feats, src_idx, dst_idx,
           W1e, W2e, ge, be, W1n, W2n, gn, bn):
    nf = node_feats[0]
    ef = edge_feats[0]

    sidx = jnp.concatenate([src_idx, jnp.zeros((E_PAD - E,), jnp.int32)])
    didx = jnp.concatenate([dst_idx, jnp.zeros((E_PAD - E,), jnp.int32)])
    sidx3 = sidx.reshape(NW, K, CHUNK)
    didx3 = didx.reshape(NW, K, CHUNK)
    ef_pad = jnp.concatenate(
        [ef, jnp.zeros((E_PAD - E, DE), jnp.float32)], axis=0)
    zeros_acc = jnp.zeros((N_ACC, DE), jnp.float32)

    gsrc, gdst, psum = _sc_gather_scatter(nf, sidx3, didx3, ef_pad, zeros_acc)

    out_e = _edge_mlp(
        gsrc, gdst, ef_pad,
        W1e[:DN], W1e[DN:2 * DN], W1e[2 * DN:],
        W2e, ge.reshape(1, DE), be.reshape(1, DE))

    nf_pad = jnp.concatenate(
        [nf, jnp.zeros((N_PAD - N, DN), jnp.float32)], axis=0)
    p0 = jnp.concatenate(
        [psum[0, :N], jnp.zeros((N_PAD - N, DE), jnp.float32)], axis=0)
    p1 = jnp.concatenate(
        [psum[1, :N], jnp.zeros((N_PAD - N, DE), jnp.float32)], axis=0)

    out_n = _node_mlp(
        nf_pad, p0, p1,
        W1n[:DN], W1n[DN:],
        W2n, gn.reshape(1, DN), bn.reshape(1, DN))

    return (out_n[:N][None], out_e[:E][None])


# speedup vs baseline: 1.1590x; 1.1590x over previous
import functools

import jax
import jax.numpy as jnp
from jax import lax
from jax.experimental import pallas as pl
from jax.experimental.pallas import tpu as pltpu, tpu_sc as plsc

N = 10000
E = 160000
DN = 256
DE = 16
LAT = 512

NC = 2
NS = 16
NW = NC * NS
CHUNK = 128
K = -(-E // (NW * CHUNK))
E_PAD = NW * K * CHUNK
STRIPE = 8 * (-(-N // (NS * 8)))
N_ACC = NS * STRIPE

BE = 512
BN = 512
N_PAD = -(-N // BN) * BN


def _sc_gather_scatter(node_tab, sidx3, didx3, edge_pad, zeros_acc):
    mesh = plsc.VectorSubcoreMesh(core_axis_name="c", subcore_axis_name="s")

    @functools.partial(
        pl.kernel,
        mesh=mesh,
        compiler_params=pltpu.CompilerParams(use_tc_tiling_on_sc=False),
        out_type=(
            jax.ShapeDtypeStruct((E_PAD, DN), node_tab.dtype),
            jax.ShapeDtypeStruct((E_PAD, DN), node_tab.dtype),
            jax.ShapeDtypeStruct((NC, N_ACC, DE), jnp.float32),
        ),
        scratch_types=[
            pltpu.VMEM((K, CHUNK), jnp.int32),
            pltpu.VMEM((K, CHUNK), jnp.int32),
            pltpu.VMEM((2, CHUNK, DN), node_tab.dtype),
            pltpu.VMEM((2, CHUNK, DN), node_tab.dtype),
            pltpu.VMEM((2, CHUNK, DE), jnp.float32),
            pltpu.VMEM((STRIPE, DE), jnp.float32),
            pltpu.VMEM_SHARED((N_ACC, DE), jnp.float32),
            pltpu.SemaphoreType.DMA,
            pltpu.SemaphoreType.DMA,
            pltpu.SemaphoreType.DMA,
            pltpu.SemaphoreType.DMA,
            pltpu.SemaphoreType.DMA,
            pltpu.SemaphoreType.DMA,
        ],
    )
    def kern(node_hbm, sidx_hbm, didx_hbm, edge_hbm, zeros_hbm,
             gsrc_hbm, gdst_hbm, psum_hbm,
             idx_s, idx_d, rows_s, rows_d, erows, sbuf, acc,
             sem_gs, sem_gd, sem_e, sem_ws, sem_wd, sem_z):
        c = lax.axis_index("c")
        s = lax.axis_index("s")
        wid = s * NC + c
        base = wid * (K * CHUNK)

        pltpu.sync_copy(sidx_hbm.at[wid], idx_s)
        pltpu.sync_copy(didx_hbm.at[wid], idx_d)
        pltpu.async_copy(zeros_hbm.at[pl.ds(s * STRIPE, STRIPE)], sbuf,
                         sem_z).wait()
        pltpu.sync_copy(sbuf, acc.at[pl.ds(s * STRIPE, STRIPE)])
        plsc.subcore_barrier()

        sems = (sem_gs, sem_gd, sem_e)

        def fire(j, p):
            off = base + j * CHUNK
            return (
                pltpu.async_copy(node_hbm.at[idx_s.at[j]], rows_s.at[p],
                                 sem_gs),
                pltpu.async_copy(node_hbm.at[idx_d.at[j]], rows_d.at[p],
                                 sem_gd),
                pltpu.async_copy(edge_hbm.at[pl.ds(off, CHUNK)], erows.at[p],
                                 sem_e),
            )

        def drain(j, p, descs):
            off = base + j * CHUNK
            descs[0].wait()
            ws = pltpu.async_copy(rows_s.at[p], gsrc_hbm.at[pl.ds(off, CHUNK)],
                                  sem_ws)
            descs[1].wait()
            wd = pltpu.async_copy(rows_d.at[p], gdst_hbm.at[pl.ds(off, CHUNK)],
                                  sem_wd)
            descs[2].wait()
            pltpu.sync_copy(erows.at[p], acc.at[idx_d.at[j]], add=True)
            return ws, wd

        @pl.loop(0, K, step=2)
        def _loop(j):
            a0 = fire(j, 0)
            a1 = fire(j + 1, 1)
            w0 = drain(j, 0, a0)
            w1 = drain(j + 1, 1, a1)
            for d in (*w0, *w1):
                d.wait()

        plsc.subcore_barrier()
        pltpu.sync_copy(acc.at[pl.ds(s * STRIPE, STRIPE)], sbuf)
        pltpu.sync_copy(sbuf, psum_hbm.at[c, pl.ds(s * STRIPE, STRIPE)])

    return kern(node_tab, sidx3, didx3, edge_pad, zeros_acc)


def _edge_mlp(gsrc, gdst, edge_pad, w1s, w1d, w1x, w2, g, b):
    def body(gs, gd, ef, w1s_r, w1d_r, w1x_r, w2_r, g_r, b_r, out):
        ef32 = ef[...]
        h = jnp.dot(gs[...], w1s_r[...], preferred_element_type=jnp.float32)
        h = h + jnp.dot(gd[...], w1d_r[...], preferred_element_type=jnp.float32)
        h = h + jnp.dot(ef32.astype(jnp.bfloat16), w1x_r[...],
                        preferred_element_type=jnp.float32)
        h = h * jax.nn.sigmoid(h)
        u = jnp.dot(h.astype(jnp.bfloat16), w2_r[...],
                    preferred_element_type=jnp.float32)
        mu = jnp.mean(u, axis=-1, keepdims=True)
        var = jnp.mean((u - mu) * (u - mu), axis=-1, keepdims=True)
        y = (u - mu) * lax.rsqrt(var + 1e-5) * g_r[...] + b_r[...]
        out[...] = y + ef32

    grid = (E_PAD // BE,)
    return pl.pallas_call(
        body,
        grid=grid,
        in_specs=[
            pl.BlockSpec((BE, DN), lambda i: (i, 0)),
            pl.BlockSpec((BE, DN), lambda i: (i, 0)),
            pl.BlockSpec((BE, DE), lambda i: (i, 0)),
            pl.BlockSpec((DN, LAT), lambda i: (0, 0)),
            pl.BlockSpec((DN, LAT), lambda i: (0, 0)),
            pl.BlockSpec((DE, LAT), lambda i: (0, 0)),
            pl.BlockSpec((LAT, DE), lambda i: (0, 0)),
            pl.BlockSpec((1, DE), lambda i: (0, 0)),
            pl.BlockSpec((1, DE), lambda i: (0, 0)),
        ],
        out_specs=pl.BlockSpec((BE, DE), lambda i: (i, 0)),
        out_shape=jax.ShapeDtypeStruct((E_PAD, DE), jnp.float32),
    )(gsrc, gdst, edge_pad, w1s, w1d, w1x, w2, g, b)


def _node_mlp(nf_pad, p0, p1, w1nn, w1ne, w2, g, b):
    def body(nf, p0_r, p1_r, w1nn_r, w1ne_r, w2_r, g_r, b_r, out):
        nf32 = nf[...]
        se = p0_r[...] + p1_r[...]
        h = jnp.dot(nf32.astype(jnp.bfloat16), w1nn_r[...],
                    preferred_element_type=jnp.float32)
        h = h + jnp.dot(se.astype(jnp.bfloat16), w1ne_r[...],
                        preferred_element_type=jnp.float32)
        h = h * jax.nn.sigmoid(h)
        u = jnp.dot(h.astype(jnp.bfloat16), w2_r[...],
                    preferred_element_type=jnp.float32)
        mu = jnp.mean(u, axis=-1, keepdims=True)
        var = jnp.mean((u - mu) * (u - mu), axis=-1, keepdims=True)
        y = (u - mu) * lax.rsqrt(var + 1e-5) * g_r[...] + b_r[...]
        out[...] = y + nf32

    grid = (N_PAD // BN,)
    return pl.pallas_call(
        body,
        grid=grid,
        in_specs=[
            pl.BlockSpec((BN, DN), lambda i: (i, 0)),
            pl.BlockSpec((BN, DE), lambda i: (i, 0)),
            pl.BlockSpec((BN, DE), lambda i: (i, 0)),
            pl.BlockSpec((DN, LAT), lambda i: (0, 0)),
            pl.BlockSpec((DE, LAT), lambda i: (0, 0)),
            pl.BlockSpec((LAT, DN), lambda i: (0, 0)),
            pl.BlockSpec((1, DN), lambda i: (0, 0)),
            pl.BlockSpec((1, DN), lambda i: (0, 0)),
        ],
        out_specs=pl.BlockSpec((BN, DN), lambda i: (i, 0)),
        out_shape=jax.ShapeDtypeStruct((N_PAD, DN), jnp.float32),
    )(nf_pad, p0, p1, w1nn, w1ne, w2, g, b)


def kernel(node_feats, edge_feats, src_idx, dst_idx,
           W1e, W2e, ge, be, W1n, W2n, gn, bn):
    nf = node_feats[0]
    ef = edge_feats[0]
    nf_bf = nf.astype(jnp.bfloat16)

    sidx = jnp.concatenate([src_idx, jnp.zeros((E_PAD - E,), jnp.int32)])
    didx = jnp.concatenate([dst_idx, jnp.zeros((E_PAD - E,), jnp.int32)])
    sidx3 = sidx.reshape(NW, K, CHUNK)
    didx3 = didx.reshape(NW, K, CHUNK)
    ef_pad = jnp.concatenate(
        [ef, jnp.zeros((E_PAD - E, DE), jnp.float32)], axis=0)
    zeros_acc = jnp.zeros((N_ACC, DE), jnp.float32)

    gsrc, gdst, psum = _sc_gather_scatter(nf_bf, sidx3, didx3, ef_pad,
                                          zeros_acc)

    bf = jnp.bfloat16
    out_e = _edge_mlp(
        gsrc, gdst, ef_pad,
        W1e[:DN].astype(bf), W1e[DN:2 * DN].astype(bf), W1e[2 * DN:].astype(bf),
        W2e.astype(bf), ge.reshape(1, DE), be.reshape(1, DE))

    nf_pad = jnp.concatenate(
        [nf, jnp.zeros((N_PAD - N, DN), jnp.float32)], axis=0)
    p0 = jnp.concatenate(
        [psum[0, :N], jnp.zeros((N_PAD - N, DE), jnp.float32)], axis=0)
    p1 = jnp.concatenate(
        [psum[1, :N], jnp.zeros((N_PAD - N, DE), jnp.float32)], axis=0)

    out_n = _node_mlp(
        nf_pad, p0, p1,
        W1n[:DN].astype(bf), W1n[DN:].astype(bf),
        W2n.astype(bf), gn.reshape(1, DN), bn.reshape(1, DN))

    return (out_n[:N][None], out_e[:E][None])
